# Initial kernel scaffold; baseline (speedup 1.0000x reference)
import functools

import jax
import jax.numpy as jnp
from jax import lax
from jax.experimental import pallas as pl
from jax.experimental.pallas import tpu as pltpu
from jax.experimental.pallas import tpu_sc as plsc

D = 32
CHUNK = 1600


@functools.lru_cache(maxsize=None)
def _build(n_rows: int):
  info = plsc.get_sparse_core_info()
  nw = info.num_cores * info.num_subcores
  assert n_rows % nw == 0
  per_w = n_rows // nw
  assert per_w % CHUNK == 0
  n_chunks = per_w // CHUNK
  mesh = plsc.VectorSubcoreMesh(core_axis_name="c", subcore_axis_name="s")

  @functools.partial(
      pl.kernel,
      mesh=mesh,
      out_type=jax.ShapeDtypeStruct((n_rows, D), jnp.float32),
      scratch_types=[
          pltpu.VMEM((CHUNK,), jnp.int32),
          pltpu.VMEM((CHUNK, D), jnp.float32),
          pltpu.SemaphoreType.DMA,
      ],
  )
  def gather_kernel(table_hbm, idx_hbm, out_hbm, idx_v, rows_v, sem):
    wid = lax.axis_index("s") * info.num_cores + lax.axis_index("c")
    w_base = wid * per_w

    def body(i, carry):
      base = w_base + i * CHUNK
      pltpu.sync_copy(idx_hbm.at[pl.ds(base, CHUNK)], idx_v)
      pltpu.async_copy(table_hbm.at[idx_v], rows_v, sem).wait()
      pltpu.sync_copy(rows_v, out_hbm.at[pl.ds(base, CHUNK)])
      return carry

    lax.fori_loop(0, n_chunks, body, 0)

  return gather_kernel


def kernel(idx, table):
  b, h = idx.shape
  flat = idx.reshape(-1).astype(jnp.int32)
  out = _build(flat.shape[0])(table, flat)
  return out.reshape(b, h, D)


# speedup vs baseline: 1.1028x; 1.1028x over previous
import functools

import jax
import jax.numpy as jnp
from jax import lax
from jax.experimental import pallas as pl
from jax.experimental.pallas import tpu as pltpu
from jax.experimental.pallas import tpu_sc as plsc

D = 32
CHUNK = 1600


@functools.lru_cache(maxsize=None)
def _build(n_rows: int):
  info = plsc.get_sparse_core_info()
  nw = info.num_cores * info.num_subcores
  assert n_rows % nw == 0
  per_w = n_rows // nw
  assert per_w % CHUNK == 0
  n_chunks = per_w // CHUNK
  mesh = plsc.VectorSubcoreMesh(core_axis_name="c", subcore_axis_name="s")

  @functools.partial(
      pl.kernel,
      mesh=mesh,
      compiler_params=pltpu.CompilerParams(use_tc_tiling_on_sc=False),
      out_type=jax.ShapeDtypeStruct((n_rows, D), jnp.float32),
      scratch_types=[
          pltpu.VMEM((CHUNK,), jnp.int32),
          pltpu.VMEM((CHUNK, D), jnp.float32),
          pltpu.SemaphoreType.DMA,
      ],
  )
  def gather_kernel(table_hbm, idx_hbm, out_hbm, idx_v, rows_v, sem):
    wid = lax.axis_index("s") * info.num_cores + lax.axis_index("c")
    w_base = wid * per_w

    def body(i, carry):
      base = w_base + i * CHUNK
      pltpu.sync_copy(idx_hbm.at[pl.ds(base, CHUNK)], idx_v)
      pltpu.async_copy(table_hbm.at[idx_v], rows_v, sem).wait()
      pltpu.sync_copy(rows_v, out_hbm.at[pl.ds(base, CHUNK)])
      return carry

    lax.fori_loop(0, n_chunks, body, 0)

  return gather_kernel


def kernel(idx, table):
  b, h = idx.shape
  flat = idx.reshape(-1).astype(jnp.int32)
  out = _build(flat.shape[0])(table, flat)
  return out.reshape(b, h, D)


# speedup vs baseline: 1.1099x; 1.0064x over previous
import functools

import jax
import jax.numpy as jnp
from jax import lax
from jax.experimental import pallas as pl
from jax.experimental.pallas import tpu as pltpu
from jax.experimental.pallas import tpu_sc as plsc

D = 32
CHUNK = 1600


@functools.lru_cache(maxsize=None)
def _build(n_rows: int):
  info = plsc.get_sparse_core_info()
  nw = info.num_cores * info.num_subcores
  assert n_rows % nw == 0
  per_w = n_rows // nw
  assert per_w % CHUNK == 0
  n_chunks = per_w // CHUNK
  mesh = plsc.VectorSubcoreMesh(core_axis_name="c", subcore_axis_name="s")

  @functools.partial(
      pl.kernel,
      mesh=mesh,
      compiler_params=pltpu.CompilerParams(use_tc_tiling_on_sc=False),
      out_type=jax.ShapeDtypeStruct((n_rows, D), jnp.float32),
      scratch_types=[
          pltpu.VMEM((per_w,), jnp.int32),
          pltpu.VMEM((CHUNK, D), jnp.float32),
          pltpu.VMEM((CHUNK, D), jnp.float32),
          pltpu.SemaphoreType.DMA,
          pltpu.SemaphoreType.DMA,
      ],
  )
  def gather_kernel(table_hbm, idx_hbm, out_hbm, idx_v, rows0, rows1, sem_g,
                    sem_o):
    wid = lax.axis_index("s") * info.num_cores + lax.axis_index("c")
    w_base = wid * per_w
    pltpu.sync_copy(idx_hbm.at[pl.ds(w_base, per_w)], idx_v)

    rows = (rows0, rows1)
    outs = [None, None]
    for i in range(n_chunks):
      s = i % 2
      if outs[s] is not None:
        outs[s].wait()
      pltpu.async_copy(
          table_hbm.at[idx_v.at[pl.ds(i * CHUNK, CHUNK)]], rows[s], sem_g
      ).wait()
      outs[s] = pltpu.async_copy(
          rows[s], out_hbm.at[pl.ds(w_base + i * CHUNK, CHUNK)], sem_o
      )
    for d in outs:
      if d is not None:
        d.wait()

  return gather_kernel


def kernel(idx, table):
  b, h = idx.shape
  flat = idx.reshape(-1).astype(jnp.int32)
  out = _build(flat.shape[0])(table, flat)
  return out.reshape(b, h, D)
